# baseline (speedup 1.0000x reference)
import jax
import jax.numpy as jnp
from jax import lax
from jax.experimental import pallas as pl
from jax.experimental.pallas import tpu as pltpu
from jax.experimental.pallas import tpu_sc as plsc

B = 16
C = 384
H = 96
W = 96
K = 90
KP = 96
L = 16
RPS = H * W // L
NC = 2
NS = 16
CCH = 16
NCH = C // 2 // CCH


def _body(x_rows, vert, out, vert_v, row_v, lane_v, idx_v, gbuf, obuf, sem):
    wid = lax.axis_index("s") * NC + lax.axis_index("c")
    b = lax.shift_right_logical(wid, 1)
    half = jnp.bitwise_and(wid, 1)

    pltpu.sync_copy(vert, vert_v)

    for t in range(KP // L):
        k = t * L + lax.iota(jnp.int32, L)
        k_eff = jnp.minimum(k, K - 1)
        base = b * (K * 2) + 2 * k_eff
        i = plsc.load_gather(vert_v, [base])
        j = plsc.load_gather(vert_v, [base + 1])
        flat = i * W + j
        row_v[pl.ds(t * L, L)] = lax.shift_right_logical(flat, 4)
        lane_v[pl.ds(t * L, L)] = jnp.bitwise_and(flat, L - 1)

    def per_chunk(ch, carry):
        c_start = half * (C // 2) + ch * CCH
        slab0 = (b * C + c_start) * RPS

        for cc in range(CCH):
            for t in range(KP // L):
                idx_v[cc, pl.ds(t * L, L)] = row_v[pl.ds(t * L, L)] + (
                    slab0 + cc * RPS
                )

        cps = [
            pltpu.async_copy(x_rows.at[idx_v.at[cc]], gbuf.at[cc], sem)
            for cc in range(CCH)
        ]
        for cp in cps:
            cp.wait()

        for cc in range(CCH):
            for t in range(KP // L):
                kvec = t * L + lax.iota(jnp.int32, L)
                lane = lane_v[pl.ds(t * L, L)]
                obuf[cc, pl.ds(t * L, L)] = plsc.load_gather(
                    gbuf.at[cc], [kvec, lane]
                )

        pltpu.sync_copy(obuf, out.at[b, pl.ds(c_start, CCH)])
        return carry

    lax.fori_loop(0, NCH, per_chunk, 0)


@jax.jit
def _sampler(x_rows, vert_flat):
    mesh = plsc.VectorSubcoreMesh(
        core_axis_name="c", subcore_axis_name="s", num_cores=NC, num_subcores=NS
    )
    f = pl.kernel(
        _body,
        out_type=jax.ShapeDtypeStruct((B, C, KP), jnp.float32),
        mesh=mesh,
        compiler_params=pltpu.CompilerParams(
            needs_layout_passes=False, use_tc_tiling_on_sc=False
        ),
        scratch_types=[
            pltpu.VMEM((B * K * 2,), jnp.int32),
            pltpu.VMEM((KP,), jnp.int32),
            pltpu.VMEM((KP,), jnp.int32),
            pltpu.VMEM((CCH, KP), jnp.int32),
            pltpu.VMEM((CCH, KP, L), jnp.float32),
            pltpu.VMEM((CCH, KP), jnp.float32),
            pltpu.SemaphoreType.DMA,
        ],
    )
    return f(x_rows, vert_flat)


def kernel(x, vertexs):
    x_rows = x.reshape(B * C * RPS, L)
    vert_flat = vertexs.astype(jnp.int32).reshape(B * K * 2)
    out = _sampler(x_rows, vert_flat)
    return out[:, :, :K]


# speedup vs baseline: 20.6969x; 20.6969x over previous
import jax
import jax.numpy as jnp
from jax import lax
from jax.experimental import pallas as pl
from jax.experimental.pallas import tpu as pltpu
from jax.experimental.pallas import tpu_sc as plsc

B = 16
C = 384
H = 96
W = 96
K = 90
KP = 96
L = 16
NC = 2
NS = 16
KH = KP // 2


def _body(table, vert, out, vert_v, idx_v, gbuf, sem):
    wid = lax.axis_index("s") * NC + lax.axis_index("c")
    b = lax.shift_right_logical(wid, 1)
    k0 = jnp.bitwise_and(wid, 1) * KH

    pltpu.sync_copy(vert, vert_v)

    for t in range(KH // L):
        k = k0 + t * L + lax.iota(jnp.int32, L)
        k_eff = jnp.minimum(k, K - 1)
        base = b * (K * 2) + 2 * k_eff
        i = plsc.load_gather(vert_v, [base])
        j = plsc.load_gather(vert_v, [base + 1])
        idx_v[pl.ds(t * L, L)] = b * (H * W) + i * W + j

    pltpu.async_copy(table.at[idx_v], gbuf, sem).wait()
    pltpu.sync_copy(gbuf, out.at[b, pl.ds(k0, KH)])


@jax.jit
def _sampler(table, vert_flat):
    mesh = plsc.VectorSubcoreMesh(
        core_axis_name="c", subcore_axis_name="s", num_cores=NC, num_subcores=NS
    )
    f = pl.kernel(
        _body,
        out_type=jax.ShapeDtypeStruct((B, KP, C), jnp.float32),
        mesh=mesh,
        compiler_params=pltpu.CompilerParams(
            needs_layout_passes=False, use_tc_tiling_on_sc=True
        ),
        scratch_types=[
            pltpu.VMEM((B * K * 2,), jnp.int32),
            pltpu.VMEM((KH,), jnp.int32),
            pltpu.VMEM((KH, C), jnp.float32),
            pltpu.SemaphoreType.DMA,
        ],
    )
    return f(table, vert_flat)


def kernel(x, vertexs):
    table = x.transpose(0, 2, 3, 1).reshape(B * H * W, C)
    vert_flat = vertexs.astype(jnp.int32).reshape(B * K * 2)
    out = _sampler(table, vert_flat)
    return out[:, :K, :].transpose(0, 2, 1)


# speedup vs baseline: 24.8355x; 1.2000x over previous
import jax
import jax.numpy as jnp
from jax import lax
from jax.experimental import pallas as pl
from jax.experimental.pallas import tpu as pltpu
from jax.experimental.pallas import tpu_sc as plsc

B = 16
C = 384
H = 96
W = 96
K = 90
L = 16
NC = 2
NS = 16
KPT = 3
NT = K // KPT


def _body(table, vert, out, vert_v, idx_v, gbuf, sem):
    wid = lax.axis_index("s") * NC + lax.axis_index("c")

    @pl.when(wid < NT)
    def _():
        pltpu.sync_copy(vert, vert_v)

        b_vec = lax.iota(jnp.int32, L)
        pair0 = b_vec * (K * 2)
        row0 = b_vec * (H * W)
        for kk in range(KPT):
            k = KPT * wid + kk
            i = plsc.load_gather(vert_v, [pair0 + 2 * k])
            j = plsc.load_gather(vert_v, [pair0 + 2 * k + 1])
            idx_v[pl.ds(kk * L, L)] = row0 + i * W + j

        pltpu.async_copy(table.at[idx_v], gbuf, sem).wait()
        for kk in range(KPT):
            pltpu.sync_copy(
                gbuf.at[pl.ds(kk * L, L)], out.at[KPT * wid + kk]
            )


@jax.jit
def _sampler(table, vert_flat):
    mesh = plsc.VectorSubcoreMesh(
        core_axis_name="c", subcore_axis_name="s", num_cores=NC, num_subcores=NS
    )
    f = pl.kernel(
        _body,
        out_type=jax.ShapeDtypeStruct((K, B, C), jnp.float32),
        mesh=mesh,
        compiler_params=pltpu.CompilerParams(
            needs_layout_passes=False, use_tc_tiling_on_sc=True
        ),
        scratch_types=[
            pltpu.VMEM((B * K * 2,), jnp.int32),
            pltpu.VMEM((KPT * L,), jnp.int32),
            pltpu.VMEM((KPT * L, C), jnp.float32),
            pltpu.SemaphoreType.DMA,
        ],
    )
    return f(table, vert_flat)


def kernel(x, vertexs):
    table = x.transpose(0, 2, 3, 1).reshape(B * H * W, C)
    vert_flat = vertexs.astype(jnp.int32).reshape(B * K * 2)
    out = _sampler(table, vert_flat)
    return out.transpose(1, 2, 0)


# speedup vs baseline: 24.8398x; 1.0002x over previous
import jax
import jax.numpy as jnp
from jax import lax
from jax.experimental import pallas as pl
from jax.experimental.pallas import tpu as pltpu
from jax.experimental.pallas import tpu_sc as plsc

B = 16
C = 384
H = 96
W = 96
K = 90
L = 16
NC = 2
NS = 16
KPT = 3
NT = K // KPT


def _body(table, vert, out, vert_v, idx_v, gbuf, sem):
    wid = lax.axis_index("s") * NC + lax.axis_index("c")

    @pl.when(wid < NT)
    def _():
        pltpu.sync_copy(vert, vert_v)

        b_vec = lax.iota(jnp.int32, L)
        pair0 = b_vec * (K * 2)
        row0 = b_vec * (H * W)
        for kk in range(KPT):
            k = KPT * wid + kk
            i = plsc.load_gather(vert_v, [pair0 + 2 * k])
            j = plsc.load_gather(vert_v, [pair0 + 2 * k + 1])
            idx_v[pl.ds(kk * L, L)] = row0 + i * W + j

        pltpu.async_copy(table.at[idx_v], gbuf, sem).wait()
        pltpu.sync_copy(gbuf, out.at[wid])


@jax.jit
def _sampler(table, vert_flat):
    mesh = plsc.VectorSubcoreMesh(
        core_axis_name="c", subcore_axis_name="s", num_cores=NC, num_subcores=NS
    )
    f = pl.kernel(
        _body,
        out_type=jax.ShapeDtypeStruct((NT, KPT * B, C), jnp.float32),
        mesh=mesh,
        compiler_params=pltpu.CompilerParams(
            needs_layout_passes=False, use_tc_tiling_on_sc=True
        ),
        scratch_types=[
            pltpu.VMEM((B * K * 2,), jnp.int32),
            pltpu.VMEM((KPT * L,), jnp.int32),
            pltpu.VMEM((KPT * L, C), jnp.float32),
            pltpu.SemaphoreType.DMA,
        ],
    )
    return f(table, vert_flat)


def kernel(x, vertexs):
    table = x.transpose(0, 2, 3, 1).reshape(B * H * W, C)
    vert_flat = vertexs.astype(jnp.int32).reshape(B * K * 2)
    out = _sampler(table, vert_flat)
    return out.reshape(K, B, C).transpose(1, 2, 0)
